# Initial kernel scaffold; baseline (speedup 1.0000x reference)
import jax
import jax.numpy as jnp
from jax.experimental import pallas as pl


def kernel(basket_embeddings, sequence_mask, pos_table, ln_gamma, ln_beta):
    raise NotImplementedError("write your pallas kernel here")


# speedup vs baseline: 4.1230x; 4.1230x over previous
import functools
import jax
import jax.numpy as jnp
from jax.experimental import pallas as pl
from jax.experimental.pallas import tpu as pltpu

EPS = 1e-12


def _tc_body(mask_ref, basket_ref, pos_ref, gamma_ref, beta_ref, out_ref):
    S = basket_ref.shape[1]
    m = mask_ref[...]
    pos = pos_ref[0:S, :]
    row0 = pos_ref[0:1, :]
    posd = (pos - row0)[None, :, :]
    emb = basket_ref[...] + row0[None, :, :] + m * posd
    mean = jnp.mean(emb, axis=-1, keepdims=True)
    c = emb - mean
    var = jnp.mean(c * c, axis=-1, keepdims=True)
    normed = c * jax.lax.rsqrt(var + EPS)
    out_ref[...] = normed * gamma_ref[...][None, :, :] + beta_ref[...][None, :, :]


@functools.partial(jax.jit, static_argnames=("block_b",))
def _tc_kernel(basket_embeddings, sequence_mask, pos_table, ln_gamma, ln_beta,
               block_b=8):
    B, S, H = basket_embeddings.shape
    grid = (B // block_b,)
    gamma2 = ln_gamma.reshape(1, H)
    beta2 = ln_beta.reshape(1, H)
    mask3 = sequence_mask.reshape(B, S, 1).astype(jnp.float32)
    return pl.pallas_call(
        _tc_body,
        grid=grid,
        in_specs=[
            pl.BlockSpec((block_b, S, 1), lambda i: (i, 0, 0)),
            pl.BlockSpec((block_b, S, H), lambda i: (i, 0, 0)),
            pl.BlockSpec(pos_table.shape, lambda i: (0, 0)),
            pl.BlockSpec((1, H), lambda i: (0, 0)),
            pl.BlockSpec((1, H), lambda i: (0, 0)),
        ],
        out_specs=pl.BlockSpec((block_b, S, H), lambda i: (i, 0, 0)),
        out_shape=jax.ShapeDtypeStruct((B, S, H), jnp.float32),
        compiler_params=pltpu.CompilerParams(
            dimension_semantics=("arbitrary",),
        ),
    )(mask3, basket_embeddings, pos_table, gamma2, beta2)


def kernel(basket_embeddings, sequence_mask, pos_table, ln_gamma, ln_beta):
    return _tc_kernel(basket_embeddings, sequence_mask, pos_table,
                      ln_gamma, ln_beta)


# speedup vs baseline: 5.5854x; 1.3547x over previous
import functools
import jax
import jax.numpy as jnp
from jax.experimental import pallas as pl
from jax.experimental.pallas import tpu as pltpu

EPS = 1e-12


def _tc_body(mask_ref, basket_ref, pos_ref, gamma_ref, beta_ref, out_ref):
    S = basket_ref.shape[1]
    m = mask_ref[...]
    pos = pos_ref[0:S, :]
    row0 = pos_ref[0:1, :]
    posd = (pos - row0)[None, :, :]
    emb = basket_ref[...] + row0[None, :, :] + m * posd
    mean = jnp.mean(emb, axis=-1, keepdims=True)
    c = emb - mean
    var = jnp.mean(c * c, axis=-1, keepdims=True)
    normed = c * jax.lax.rsqrt(var + EPS)
    out_ref[...] = normed * gamma_ref[...][None, :, :] + beta_ref[...][None, :, :]


@functools.partial(jax.jit, static_argnames=("block_b",))
def _tc_kernel(basket_embeddings, sequence_mask, pos_table, ln_gamma, ln_beta,
               block_b=8):
    B, S, H = basket_embeddings.shape
    grid = (B // block_b,)
    gamma2 = ln_gamma.reshape(1, H)
    beta2 = ln_beta.reshape(1, H)
    mask3 = sequence_mask.reshape(B, S, 1).astype(jnp.float32)
    return pl.pallas_call(
        _tc_body,
        grid=grid,
        in_specs=[
            pl.BlockSpec((block_b, S, 1), lambda i: (i, 0, 0)),
            pl.BlockSpec((block_b, S, H), lambda i: (i, 0, 0)),
            pl.BlockSpec(pos_table.shape, lambda i: (0, 0)),
            pl.BlockSpec((1, H), lambda i: (0, 0)),
            pl.BlockSpec((1, H), lambda i: (0, 0)),
        ],
        out_specs=pl.BlockSpec((block_b, S, H), lambda i: (i, 0, 0)),
        out_shape=jax.ShapeDtypeStruct((B, S, H), jnp.float32),
        compiler_params=pltpu.CompilerParams(
            dimension_semantics=("arbitrary",),
        ),
    )(mask3, basket_embeddings, pos_table, gamma2, beta2)


def kernel(basket_embeddings, sequence_mask, pos_table, ln_gamma, ln_beta):
    return _tc_kernel(basket_embeddings, sequence_mask, pos_table,
                      ln_gamma, ln_beta, block_b=32)


# speedup vs baseline: 5.8005x; 1.0385x over previous
import functools
import jax
import jax.numpy as jnp
from jax.experimental import pallas as pl
from jax.experimental.pallas import tpu as pltpu

EPS = 1e-12


def _tc_body(mask_ref, basket_ref, pos_ref, gamma_ref, beta_ref, out_ref):
    S = basket_ref.shape[1]
    m = mask_ref[...]
    pos = pos_ref[0:S, :]
    row0 = pos_ref[0:1, :]
    posd = (pos - row0)[None, :, :]
    emb = basket_ref[...] + row0[None, :, :] + m * posd
    mean = jnp.mean(emb, axis=-1, keepdims=True)
    c = emb - mean
    var = jnp.mean(c * c, axis=-1, keepdims=True)
    normed = c * jax.lax.rsqrt(var + EPS)
    out_ref[...] = normed * gamma_ref[...][None, :, :] + beta_ref[...][None, :, :]


@functools.partial(jax.jit, static_argnames=("block_b",))
def _tc_kernel(basket_embeddings, sequence_mask, pos_table, ln_gamma, ln_beta,
               block_b=8):
    B, S, H = basket_embeddings.shape
    grid = (B // block_b,)
    gamma2 = ln_gamma.reshape(1, H)
    beta2 = ln_beta.reshape(1, H)
    mask3 = sequence_mask.reshape(B, S, 1).astype(jnp.float32)
    return pl.pallas_call(
        _tc_body,
        grid=grid,
        in_specs=[
            pl.BlockSpec((block_b, S, 1), lambda i: (i, 0, 0)),
            pl.BlockSpec((block_b, S, H), lambda i: (i, 0, 0)),
            pl.BlockSpec(pos_table.shape, lambda i: (0, 0)),
            pl.BlockSpec((1, H), lambda i: (0, 0)),
            pl.BlockSpec((1, H), lambda i: (0, 0)),
        ],
        out_specs=pl.BlockSpec((block_b, S, H), lambda i: (i, 0, 0)),
        out_shape=jax.ShapeDtypeStruct((B, S, H), jnp.float32),
        compiler_params=pltpu.CompilerParams(
            dimension_semantics=("arbitrary",),
        ),
    )(mask3, basket_embeddings, pos_table, gamma2, beta2)


def kernel(basket_embeddings, sequence_mask, pos_table, ln_gamma, ln_beta):
    return _tc_kernel(basket_embeddings, sequence_mask, pos_table,
                      ln_gamma, ln_beta, block_b=64)


# speedup vs baseline: 12.0808x; 2.0827x over previous
import functools
import jax
import jax.numpy as jnp
from jax.experimental import pallas as pl
from jax.experimental.pallas import tpu as pltpu

EPS = 1e-12


def _tc_body(mask_ref, basket_ref, pos_ref, gamma_ref, beta_ref, out_ref):
    Bb, S, H = basket_ref.shape
    m2 = mask_ref[...]
    m3 = jax.lax.broadcast_in_dim(m2, (Bb, S, H), (0, 1))
    pos = pos_ref[0:S, :]
    row0 = pos_ref[0:1, :]
    posd = (pos - row0)[None, :, :]
    emb = basket_ref[...] + row0[None, :, :] + m3 * posd
    emb2 = emb.reshape(Bb * S, H)
    ones = jnp.full((H, H), 1.0 / H, dtype=jnp.float32)
    mean = jnp.dot(emb2, ones, preferred_element_type=jnp.float32)
    msq = jnp.dot(emb2 * emb2, ones, preferred_element_type=jnp.float32)
    var = msq - mean * mean
    c = emb2 - mean
    normed = c * jax.lax.rsqrt(var + EPS)
    out = normed * gamma_ref[...] + beta_ref[...]
    out_ref[...] = out.reshape(Bb, S, H)


@functools.partial(jax.jit, static_argnames=("block_b",))
def _tc_kernel(basket_embeddings, sequence_mask, pos_table, ln_gamma, ln_beta,
               block_b=64):
    B, S, H = basket_embeddings.shape
    grid = (B // block_b,)
    gamma2 = ln_gamma.reshape(1, H)
    beta2 = ln_beta.reshape(1, H)
    maskf = sequence_mask.astype(jnp.float32)
    return pl.pallas_call(
        _tc_body,
        grid=grid,
        in_specs=[
            pl.BlockSpec((block_b, S), lambda i: (i, 0)),
            pl.BlockSpec((block_b, S, H), lambda i: (i, 0, 0)),
            pl.BlockSpec(pos_table.shape, lambda i: (0, 0)),
            pl.BlockSpec((1, H), lambda i: (0, 0)),
            pl.BlockSpec((1, H), lambda i: (0, 0)),
        ],
        out_specs=pl.BlockSpec((block_b, S, H), lambda i: (i, 0, 0)),
        out_shape=jax.ShapeDtypeStruct((B, S, H), jnp.float32),
        compiler_params=pltpu.CompilerParams(
            dimension_semantics=("arbitrary",),
        ),
    )(maskf, basket_embeddings, pos_table, gamma2, beta2)


def kernel(basket_embeddings, sequence_mask, pos_table, ln_gamma, ln_beta):
    return _tc_kernel(basket_embeddings, sequence_mask, pos_table,
                      ln_gamma, ln_beta, block_b=64)


# speedup vs baseline: 12.1749x; 1.0078x over previous
import functools
import jax
import jax.numpy as jnp
from jax.experimental import pallas as pl
from jax.experimental.pallas import tpu as pltpu

EPS = 1e-12


def _tc_body(mask_ref, basket_ref, pos_ref, gamma_ref, beta_ref, out_ref):
    Bb, S, H = basket_ref.shape
    m2 = mask_ref[...]
    m3 = jax.lax.broadcast_in_dim(m2, (Bb, S, H), (0, 1))
    pos = pos_ref[0:S, :]
    row0 = pos_ref[0:1, :]
    posd = (pos - row0)[None, :, :]
    emb = basket_ref[...] + row0[None, :, :] + m3 * posd
    emb2 = emb.reshape(Bb * S, H)
    ones = jnp.full((H, H), 1.0 / H, dtype=jnp.float32)
    mean = jnp.dot(emb2, ones, preferred_element_type=jnp.float32)
    msq = jnp.dot(emb2 * emb2, ones, preferred_element_type=jnp.float32)
    var = msq - mean * mean
    c = emb2 - mean
    normed = c * jax.lax.rsqrt(var + EPS)
    out = normed * gamma_ref[...] + beta_ref[...]
    out_ref[...] = out.reshape(Bb, S, H)


@functools.partial(jax.jit, static_argnames=("block_b",))
def _tc_kernel(basket_embeddings, sequence_mask, pos_table, ln_gamma, ln_beta,
               block_b=64):
    B, S, H = basket_embeddings.shape
    grid = (B // block_b,)
    gamma2 = ln_gamma.reshape(1, H)
    beta2 = ln_beta.reshape(1, H)
    maskf = sequence_mask.astype(jnp.float32)
    return pl.pallas_call(
        _tc_body,
        grid=grid,
        in_specs=[
            pl.BlockSpec((block_b, S), lambda i: (i, 0)),
            pl.BlockSpec((block_b, S, H), lambda i: (i, 0, 0)),
            pl.BlockSpec(pos_table.shape, lambda i: (0, 0)),
            pl.BlockSpec((1, H), lambda i: (0, 0)),
            pl.BlockSpec((1, H), lambda i: (0, 0)),
        ],
        out_specs=pl.BlockSpec((block_b, S, H), lambda i: (i, 0, 0)),
        out_shape=jax.ShapeDtypeStruct((B, S, H), jnp.float32),
        compiler_params=pltpu.CompilerParams(
            dimension_semantics=("arbitrary",),
        ),
    )(maskf, basket_embeddings, pos_table, gamma2, beta2)


def kernel(basket_embeddings, sequence_mask, pos_table, ln_gamma, ln_beta):
    return _tc_kernel(basket_embeddings, sequence_mask, pos_table,
                      ln_gamma, ln_beta, block_b=128)
